# Initial kernel scaffold; baseline (speedup 1.0000x reference)
import jax
import jax.numpy as jnp
from jax.experimental import pallas as pl


def kernel(scores, gt_boxes, metadata):
    raise NotImplementedError("write your pallas kernel here")


# speedup vs baseline: 6.2814x; 6.2814x over previous
import numpy as np
import jax
import jax.numpy as jnp
from jax.experimental import pallas as pl
from jax.experimental.pallas import tpu as pltpu

_STRIDE = 16
_NEG_OVERLAP = 0.3
_POS_OVERLAP = 0.7
_RPN_BATCHSIZE = 256
_NUM_FG = 128
_FH = _FW = 64
_G = 100
_A = _FH * _FW * 9
_C = 128
_R = _A // _C


def _np_base_anchors(base_size=16, ratios=(0.5, 1.0, 2.0), scales=(8, 16, 32)):
    base = np.array([1, 1, base_size, base_size], dtype=np.float32) - 1
    w = base[2] - base[0] + 1
    h = base[3] - base[1] + 1
    x_ctr = base[0] + 0.5 * (w - 1)
    y_ctr = base[1] + 0.5 * (h - 1)
    size = w * h
    anchors = []
    for r in ratios:
        size_r = size / r
        ws = np.round(np.sqrt(size_r))
        hs = np.round(ws * r)
        for s in scales:
            ws2 = ws * s
            hs2 = hs * s
            anchors.append([x_ctr - 0.5 * (ws2 - 1), y_ctr - 0.5 * (hs2 - 1),
                            x_ctr + 0.5 * (ws2 - 1), y_ctr + 0.5 * (hs2 - 1)])
    return np.array(anchors, dtype=np.float32)


def _np_all_anchors(fh, fw, stride, base):
    sx = np.arange(fw, dtype=np.float32) * stride
    sy = np.arange(fh, dtype=np.float32) * stride
    sx, sy = np.meshgrid(sx, sy)
    shifts = np.stack([sx.ravel(), sy.ravel(), sx.ravel(), sy.ravel()],
                      axis=1).astype(np.float32)
    all_a = base[None, :, :] + shifts[:, None, :]
    return all_a.reshape(-1, 4)


_ANCHORS = _np_all_anchors(_FH, _FW, _STRIDE, _np_base_anchors())
_AX1 = _ANCHORS[:, 0].reshape(_R, _C)
_AY1 = _ANCHORS[:, 1].reshape(_R, _C)
_AX2 = _ANCHORS[:, 2].reshape(_R, _C)
_AY2 = _ANCHORS[:, 3].reshape(_R, _C)

_bk = jax.random.key(42)
_kf, _kb = jax.random.split(_bk)
_RND_F = np.asarray(jax.random.uniform(_kf, (_A,)), dtype=np.float32)
_RND_B = np.asarray(jax.random.uniform(_kb, (_A,)), dtype=np.float32)
_ORD_F = np.argsort(_RND_F, kind="stable").astype(np.int32)
_ORD_B = np.argsort(_RND_B, kind="stable").astype(np.int32)
_SV_F = _RND_F[_ORD_F].reshape(_R, _C)
_SV_B = _RND_B[_ORD_B].reshape(_R, _C)
_SI_F = _ORD_F.reshape(_R, _C)
_SI_B = _ORD_B.reshape(_R, _C)
_RF2 = _RND_F.reshape(_R, _C)
_RB2 = _RND_B.reshape(_R, _C)


def _body(gt_ref, meta_ref,
          ax1_ref, ay1_ref, ax2_ref, ay2_ref,
          rf_ref, svf_ref, sif_ref, rb_ref, svb_ref, sib_ref,
          lab_ref, dx_ref, dy_ref, dw_ref, dh_ref,
          colarg_ref):
    ax1 = ax1_ref[...]
    ay1 = ay1_ref[...]
    ax2 = ax2_ref[...]
    ay2 = ay2_ref[...]
    m_h = meta_ref[0]
    m_w = meta_ref[1]
    inside = ((ax1 >= 0.0) & (ay1 >= 0.0) & (ax2 < m_w) & (ay2 < m_h))
    aw = ax2 - ax1 + 1.0
    ah = ay2 - ay1 + 1.0
    aarea = aw * ah
    pos = (jax.lax.broadcasted_iota(jnp.int32, (_R, _C), 0) * _C
           + jax.lax.broadcasted_iota(jnp.int32, (_R, _C), 1))

    def j_body(j, carry):
        bv, bgw, bgh, bgcx, bgcy = carry
        gx1 = gt_ref[j, 0]
        gy1 = gt_ref[j, 1]
        gx2 = gt_ref[j, 2]
        gy2 = gt_ref[j, 3]
        gw = gx2 - gx1 + 1.0
        gh = gy2 - gy1 + 1.0
        garea = gw * gh
        gcx = gx1 + 0.5 * gw
        gcy = gy1 + 0.5 * gh
        iw = jnp.minimum(ax2, gx2) - jnp.maximum(ax1, gx1) + 1.0
        ih = jnp.minimum(ay2, gy2) - jnp.maximum(ay1, gy1) + 1.0
        iw = jnp.maximum(iw, 0.0)
        ih = jnp.maximum(ih, 0.0)
        inter = iw * ih
        union = aarea + garea - inter
        iou = inter / union
        masked = jnp.where(inside, iou, -1.0)
        c = masked > bv
        bv = jnp.where(c, masked, bv)
        bgw = jnp.where(c, gw, bgw)
        bgh = jnp.where(c, gh, bgh)
        bgcx = jnp.where(c, gcx, bgcx)
        bgcy = jnp.where(c, gcy, bgcy)
        cm = jnp.max(masked)
        colarg_ref[j] = jnp.min(jnp.where(masked == cm, pos, _A))
        return (bv, bgw, bgh, bgcx, bgcy)

    ninf = jnp.full((_R, _C), -jnp.inf, jnp.float32)
    one = jnp.ones((_R, _C), jnp.float32)
    bv, bgw, bgh, bgcx, bgcy = jax.lax.fori_loop(
        0, _G, j_body, (ninf, one, one, one, one))

    def h_body(j, hit):
        return hit | (pos == colarg_ref[j]).astype(jnp.int32)

    hit = jax.lax.fori_loop(0, _G, h_body, jnp.zeros((_R, _C), jnp.int32))

    labels = jnp.where(inside & (bv < _NEG_OVERLAP), 0.0, -1.0)
    labels = jnp.where(hit > 0, 1.0, labels)
    labels = jnp.where(inside & (bv >= _POS_OVERLAP), 1.0, labels)
    labels = jnp.where(inside, labels, -1.0)

    def subsample(flag, rnd, sv_ref, si_ref, target):
        sv = sv_ref[...]
        si = si_ref[...]

        def fetch(m):
            sel = pos == m
            v = jnp.sum(jnp.where(sel, sv, 0.0))
            t = jnp.sum(jnp.where(sel, si, 0))
            return v, t

        def bs_body(_, lohi):
            lo, hi = lohi
            done = lo >= hi
            mid = (lo + hi) // 2
            v, t = fetch(mid)
            le = (rnd < v) | ((rnd == v) & (pos <= t))
            cnt = jnp.sum((flag & le).astype(jnp.int32))
            c = cnt >= target
            lo2 = jnp.where(done, lo, jnp.where(c, lo, mid + 1))
            hi2 = jnp.where(done, hi, jnp.where(c, mid, hi))
            return (lo2, hi2)

        lo, _ = jax.lax.fori_loop(0, 16, bs_body,
                                  (jnp.int32(0), jnp.int32(_A)))
        v, t = fetch(jnp.minimum(lo, _A - 1))
        keep_all = lo >= _A
        kept = flag & (keep_all | (rnd < v) | ((rnd == v) & (pos <= t)))
        return kept

    fg = labels == 1.0
    n_fg = jnp.sum(fg.astype(jnp.int32))
    kept_f = subsample(fg, rf_ref[...], svf_ref, sif_ref, jnp.int32(_NUM_FG))
    labels = jnp.where(fg & (~kept_f), -1.0, labels)
    num_bg = _RPN_BATCHSIZE - jnp.minimum(n_fg, _NUM_FG)
    bg = labels == 0.0
    kept_b = subsample(bg, rb_ref[...], svb_ref, sib_ref, num_bg)
    labels = jnp.where(bg & (~kept_b), -1.0, labels)

    acx = ax1 + 0.5 * aw
    acy = ay1 + 0.5 * ah
    dx = (bgcx - acx) / aw
    dy = (bgcy - acy) / ah
    dw = jnp.log(bgw / aw)
    dh = jnp.log(bgh / ah)

    lab_ref[...] = labels
    dx_ref[...] = jnp.where(inside, dx, 0.0)
    dy_ref[...] = jnp.where(inside, dy, 0.0)
    dw_ref[...] = jnp.where(inside, dw, 0.0)
    dh_ref[...] = jnp.where(inside, dh, 0.0)


def kernel(scores, gt_boxes, metadata):
    del scores
    f32 = jnp.float32
    out_shapes = [jax.ShapeDtypeStruct((_R, _C), f32) for _ in range(5)]
    smem = pl.BlockSpec(memory_space=pltpu.SMEM)
    labels, dx, dy, dw, dh = pl.pallas_call(
        _body,
        out_shape=out_shapes,
        in_specs=[smem, smem] + [pl.BlockSpec((_R, _C), lambda: (0, 0))] * 10,
        out_specs=[pl.BlockSpec((_R, _C), lambda: (0, 0))] * 5,
        scratch_shapes=[pltpu.SMEM((_G,), jnp.int32)],
    )(gt_boxes, metadata,
      jnp.asarray(_AX1), jnp.asarray(_AY1), jnp.asarray(_AX2), jnp.asarray(_AY2),
      jnp.asarray(_RF2), jnp.asarray(_SV_F), jnp.asarray(_SI_F),
      jnp.asarray(_RB2), jnp.asarray(_SV_B), jnp.asarray(_SI_B))
    cols = [labels, dx, dy, dw, dh]
    return jnp.stack([c.reshape(-1) for c in cols], axis=1)
